# Initial kernel scaffold; baseline (speedup 1.0000x reference)
import jax
import jax.numpy as jnp
from jax.experimental import pallas as pl


def kernel(h, edge_index, Wq, Wk, Wv, Wo, ln1_scale, ln1_bias, ln2_scale, ln2_bias, W1, b1, W2, b2):
    raise NotImplementedError("write your pallas kernel here")


# speedup vs baseline: 3.5988x; 3.5988x over previous
import functools

import jax
import jax.numpy as jnp
import numpy as np
from jax import lax
from jax.experimental import pallas as pl
from jax.experimental.pallas import tpu as pltpu
from jax.experimental.pallas import tpu_sc as plsc

N = 10000
E = 160000
D = 256
H = 8
DK = 32
DFF = 1024

NPAD = 10240
HALF = NPAD // 2
NSUB = 16
NCORE = 2
NW = NSUB * NCORE
E_PAD = 163840
EPW = E_PAD // NW
B1 = 64
NBLK1 = EPW // B1
B2 = 512
NBLK2 = E_PAD // B2
FS = 16
DUMP = HALF
NSLAB = 2 * NSUB + 1
SLABW = B1 * FS
BLKW = NSLAB * SLABW
NBLK_G = E_PAD // B1


def _pass1_body(k_hbm, q_hbm, v_hbm, src_hbm, dst_hbm, wv_hbm,
                s_v, d_v, dg_v, k_v, q_v, v_v, cb_v, sem1, sem2, sem3):
    c = lax.axis_index("c")
    s = lax.axis_index("s")
    wid = c * NSUB + s
    zero16 = jnp.zeros((16,), jnp.float32)
    lane = lax.iota(jnp.int32, 16)
    rs = np.float32(1.0 / np.sqrt(DK))
    ebase = wid * EPW

    def eblock(i, carry):
        base = ebase + i * B1
        pltpu.sync_copy(src_hbm.at[pl.ds(base, B1)], s_v)
        pltpu.sync_copy(dst_hbm.at[pl.ds(base, B1)], d_v)
        for j in range(B1 // 16):
            dv = d_v[pl.ds(j * 16, 16)]
            dg_v[pl.ds(j * 16, 16)] = jnp.minimum(dv, N - 1)
        cp1 = pltpu.async_copy(k_hbm.at[s_v], k_v, sem1)
        cp2 = pltpu.async_copy(q_hbm.at[dg_v], q_v, sem2)
        cp3 = pltpu.async_copy(v_hbm.at[s_v], v_v, sem3)
        cp1.wait()
        cp2.wait()
        cp3.wait()

        def edge(e, ecarry):
            srow = zero16
            for hh in range(H):
                k0 = k_v[e, pl.ds(hh * DK, 16)]
                k1 = k_v[e, pl.ds(hh * DK + 16, 16)]
                q0 = q_v[e, pl.ds(hh * DK, 16)]
                q1 = q_v[e, pl.ds(hh * DK + 16, 16)]
                p = k0 * q0 + k1 * q1
                for shift in (8, 4, 2, 1):
                    p = p + p.at[lane ^ shift].get(mode="promise_in_bounds")
                wm = jnp.exp(p * rs)
                v0 = v_v[e, pl.ds(hh * DK, 16)]
                v1 = v_v[e, pl.ds(hh * DK + 16, 16)]
                cb_v[pl.ds((2 * hh) * SLABW + e * FS, 16)] = v0 * wm
                cb_v[pl.ds((2 * hh + 1) * SLABW + e * FS, 16)] = v1 * wm
                srow = srow + jnp.where(lane == hh, wm, jnp.float32(0.0))
            cb_v[pl.ds(2 * NSUB * SLABW + e * FS, 16)] = srow
            return ecarry
        lax.fori_loop(0, B1, edge, 0)

        g = wid * NBLK1 + i
        pltpu.sync_copy(cb_v, wv_hbm.at[pl.ds(g * BLKW, BLKW)])
        return carry
    lax.fori_loop(0, NBLK1, eblock, 0)


def _pass2_body(dst_hbm, wv_hbm, anum_hbm, ssum_hbm,
                d_v, wv_v, ex_v, acc_v, ssa_v, sem1, sem2):
    c = lax.axis_index("c")
    s = lax.axis_index("s")
    hh = s // 2
    lane = lax.iota(jnp.int32, 16)
    zero16 = jnp.zeros((16,), jnp.float32)
    even = s % 2 == 0

    def zacc(r, carry):
        acc_v[pl.ds(r * 16, 16)] = zero16
        return carry
    lax.fori_loop(0, (HALF + 8) * FS // 16, zacc, 0)

    def zss(r, carry):
        ssa_v[pl.ds(r * 16, 16)] = zero16
        return carry
    lax.fori_loop(0, (HALF + 16) // 16, zss, 0)

    def eblock(i, carry):
        base = i * B2
        g0 = i * (B2 // B1)
        pltpu.sync_copy(dst_hbm.at[pl.ds(base, B2)], d_v)
        cps = []
        for t in range(B2 // B1):
            cps.append(pltpu.async_copy(
                wv_hbm.at[pl.ds((g0 + t) * BLKW + s * SLABW, SLABW)],
                wv_v.at[pl.ds(t * SLABW, SLABW)], sem1))

        @pl.when(even)
        def _():
            for t in range(B2 // B1):
                pltpu.async_copy(
                    wv_hbm.at[pl.ds((g0 + t) * BLKW + 2 * NSUB * SLABW,
                                    SLABW)],
                    ex_v.at[pl.ds(t * SLABW, SLABW)], sem2).wait()
        for cp in cps:
            cp.wait()

        for j in range(B2 // 16):
            d16 = d_v[pl.ds(j * 16, 16)]
            lv = d16 - c * HALF
            inr = (lv >= 0) & (lv < HALF)
            lidx = jnp.where(inr, lv, DUMP)
            lbase = lidx * FS
            for f in range(FS):
                vcol = plsc.load_gather(
                    wv_v, [lane * FS + (j * 16 * FS + f)])
                plsc.addupdate_scatter(acc_v, [lbase + f], vcol)

            @pl.when(even)
            def _():
                w16 = plsc.load_gather(
                    ex_v, [lane * FS + (j * 16 * FS + hh)])
                plsc.addupdate_scatter(ssa_v, [lidx], w16)
        return carry
    lax.fori_loop(0, NBLK2, eblock, 0)

    wid = c * NSUB + s
    pltpu.sync_copy(acc_v.at[pl.ds(0, HALF * FS)],
                    anum_hbm.at[pl.ds(wid * HALF * FS, HALF * FS)])

    @pl.when(even)
    def _():
        off = (c * H + hh) * HALF
        pltpu.sync_copy(ssa_v.at[pl.ds(0, HALF)],
                        ssum_hbm.at[pl.ds(off, HALF)])


def _edge_phase(q, k, v, src_p, dst_p):
    mesh = plsc.VectorSubcoreMesh(core_axis_name="c", subcore_axis_name="s")
    cparams = pltpu.CompilerParams(needs_layout_passes=False)
    wv_flat = functools.partial(
        pl.kernel,
        mesh=mesh,
        compiler_params=cparams,
        out_type=jax.ShapeDtypeStruct((NBLK_G * BLKW,), jnp.float32),
        scratch_types=[
            pltpu.VMEM((B1,), jnp.int32),
            pltpu.VMEM((B1,), jnp.int32),
            pltpu.VMEM((B1,), jnp.int32),
            pltpu.VMEM((B1, D), jnp.float32),
            pltpu.VMEM((B1, D), jnp.float32),
            pltpu.VMEM((B1, D), jnp.float32),
            pltpu.VMEM((BLKW,), jnp.float32),
            pltpu.SemaphoreType.DMA,
            pltpu.SemaphoreType.DMA,
            pltpu.SemaphoreType.DMA,
        ],
    )(_pass1_body)(k, q, v, src_p, dst_p)

    anum_flat, ssum_flat = functools.partial(
        pl.kernel,
        mesh=mesh,
        compiler_params=cparams,
        out_type=[
            jax.ShapeDtypeStruct((NW * HALF * FS,), jnp.float32),
            jax.ShapeDtypeStruct((NCORE * H * HALF,), jnp.float32),
        ],
        scratch_types=[
            pltpu.VMEM((B2,), jnp.int32),
            pltpu.VMEM((B2 * FS,), jnp.float32),
            pltpu.VMEM((B2 * FS,), jnp.float32),
            pltpu.VMEM(((HALF + 8) * FS,), jnp.float32),
            pltpu.VMEM((HALF + 16,), jnp.float32),
            pltpu.SemaphoreType.DMA,
            pltpu.SemaphoreType.DMA,
        ],
    )(_pass2_body)(dst_p, wv_flat)
    return anum_flat, ssum_flat


def _qkv_body(h_ref, wq_ref, wk_ref, wv_ref, q_ref, k_ref, v_ref):
    hb = h_ref[...]
    q_ref[...] = jnp.dot(hb, wq_ref[...], preferred_element_type=jnp.float32)
    k_ref[...] = jnp.dot(hb, wk_ref[...], preferred_element_type=jnp.float32)
    v_ref[...] = jnp.dot(hb, wv_ref[...], preferred_element_type=jnp.float32)


def _ln(x, scale, bias):
    mu = jnp.mean(x, axis=-1, keepdims=True)
    var = jnp.mean((x - mu) ** 2, axis=-1, keepdims=True)
    return (x - mu) / jnp.sqrt(var + 1e-5) * scale + bias


def _post_body(anum_ref, ssum_ref, h_ref, exp_ref, wo_ref, l1s_ref, l1b_ref,
               w1_ref, b1_ref, w2_ref, b2_ref, l2s_ref, l2b_ref, out_ref):
    denom = jnp.dot(ssum_ref[...], exp_ref[...],
                    preferred_element_type=jnp.float32) + 1e-9
    a = anum_ref[...] / denom
    o = jnp.dot(a, wo_ref[...], preferred_element_type=jnp.float32)
    h1 = _ln(h_ref[...] + o, l1s_ref[...], l1b_ref[...])
    f = jnp.maximum(
        jnp.dot(h1, w1_ref[...], preferred_element_type=jnp.float32)
        + b1_ref[...], 0.0)
    f2 = jnp.dot(f, w2_ref[...], preferred_element_type=jnp.float32) + b2_ref[...]
    out_ref[...] = _ln(h1 + f2, l2s_ref[...], l2b_ref[...])


BLK = 1000
GRID = N // BLK


def kernel(h, edge_index, Wq, Wk, Wv, Wo, ln1_scale, ln1_bias, ln2_scale,
           ln2_bias, W1, b1, W2, b2):
    q, k, v = pl.pallas_call(
        _qkv_body,
        grid=(GRID,),
        in_specs=[
            pl.BlockSpec((BLK, D), lambda i: (i, 0)),
            pl.BlockSpec((D, D), lambda i: (0, 0)),
            pl.BlockSpec((D, D), lambda i: (0, 0)),
            pl.BlockSpec((D, D), lambda i: (0, 0)),
        ],
        out_specs=[
            pl.BlockSpec((BLK, D), lambda i: (i, 0)),
            pl.BlockSpec((BLK, D), lambda i: (i, 0)),
            pl.BlockSpec((BLK, D), lambda i: (i, 0)),
        ],
        out_shape=[
            jax.ShapeDtypeStruct((N, D), jnp.float32),
            jax.ShapeDtypeStruct((N, D), jnp.float32),
            jax.ShapeDtypeStruct((N, D), jnp.float32),
        ],
    )(h, Wq, Wk, Wv)

    src_p = jnp.pad(edge_index[0], (0, E_PAD - E))
    dst_p = jnp.pad(edge_index[1], (0, E_PAD - E), constant_values=1 << 20)

    anum_flat, ssum_flat = _edge_phase(q, k, v, src_p, dst_p)
    anum = (anum_flat.reshape(NCORE, NSUB, HALF, FS)
            .transpose(0, 2, 1, 3).reshape(NPAD, D)[:N])
    ssum = (ssum_flat.reshape(NCORE, H, HALF)
            .transpose(0, 2, 1).reshape(NPAD, H)[:N])

    expand = (jnp.arange(D, dtype=jnp.int32)[None, :] // DK
              == jnp.arange(H, dtype=jnp.int32)[:, None]).astype(jnp.float32)

    out = pl.pallas_call(
        _post_body,
        grid=(GRID,),
        in_specs=[
            pl.BlockSpec((BLK, D), lambda i: (i, 0)),
            pl.BlockSpec((BLK, H), lambda i: (i, 0)),
            pl.BlockSpec((BLK, D), lambda i: (i, 0)),
            pl.BlockSpec((H, D), lambda i: (0, 0)),
            pl.BlockSpec((D, D), lambda i: (0, 0)),
            pl.BlockSpec((1, D), lambda i: (0, 0)),
            pl.BlockSpec((1, D), lambda i: (0, 0)),
            pl.BlockSpec((D, DFF), lambda i: (0, 0)),
            pl.BlockSpec((1, DFF), lambda i: (0, 0)),
            pl.BlockSpec((DFF, D), lambda i: (0, 0)),
            pl.BlockSpec((1, D), lambda i: (0, 0)),
            pl.BlockSpec((1, D), lambda i: (0, 0)),
            pl.BlockSpec((1, D), lambda i: (0, 0)),
        ],
        out_specs=pl.BlockSpec((BLK, D), lambda i: (i, 0)),
        out_shape=jax.ShapeDtypeStruct((N, D), jnp.float32),
    )(anum, ssum, h, expand, Wo,
      ln1_scale.reshape(1, D), ln1_bias.reshape(1, D),
      W1, b1.reshape(1, DFF), W2, b2.reshape(1, D),
      ln2_scale.reshape(1, D), ln2_bias.reshape(1, D))
    return out


# speedup vs baseline: 5.0838x; 1.4126x over previous
import functools

import jax
import jax.numpy as jnp
import numpy as np
from jax import lax
from jax.experimental import pallas as pl
from jax.experimental.pallas import tpu as pltpu
from jax.experimental.pallas import tpu_sc as plsc

N = 10000
E = 160000
D = 256
H = 8
DK = 32
DFF = 1024

NPAD = 10240
HALF = NPAD // 2
NSUB = 16
NCORE = 2
NW = NSUB * NCORE
E_PAD = 163840
EPW = E_PAD // NW
B1 = 64
NBLK1 = EPW // B1
B2 = 512
NBLK2 = E_PAD // B2
FS = 16
DUMP = HALF
NSLAB = 2 * NSUB + 1
SLABW = B1 * FS
BLKW = NSLAB * SLABW
NBLK_G = E_PAD // B1


def _pass1_body(k_hbm, q_hbm, v_hbm, src_hbm, dst_hbm, wv_hbm,
                s_v, d_v, dg_v, k_v, q_v, v_v, cb_v, sem1, sem2, sem3, semw):
    c = lax.axis_index("c")
    s = lax.axis_index("s")
    wid = c * NSUB + s
    zero16 = jnp.zeros((16,), jnp.float32)
    lane = lax.iota(jnp.int32, 16)
    rs = np.float32(1.0 / np.sqrt(DK))
    ebase = wid * EPW

    pltpu.sync_copy(src_hbm.at[pl.ds(ebase, EPW)], s_v)
    pltpu.sync_copy(dst_hbm.at[pl.ds(ebase, EPW)], d_v)

    def clamp(r, carry):
        dv = d_v[pl.ds(r * 16, 16)]
        dg_v[pl.ds(r * 16, 16)] = jnp.minimum(dv, N - 1)
        return carry
    lax.fori_loop(0, EPW // 16, clamp, 0)

    def eblock(i, carry):
        base = ebase + i * B1
        cp1 = pltpu.async_copy(k_hbm.at[s_v.at[pl.ds(i * B1, B1)]],
                               k_v, sem1)
        cp2 = pltpu.async_copy(q_hbm.at[dg_v.at[pl.ds(i * B1, B1)]],
                               q_v, sem2)
        cp3 = pltpu.async_copy(v_hbm.at[s_v.at[pl.ds(i * B1, B1)]],
                               v_v, sem3)
        cp1.wait()
        cp2.wait()
        cp3.wait()
        @pl.when(i > 0)
        def _():
            pltpu.make_async_copy(
                wv_hbm.at[pl.ds(0, BLKW)], cb_v, semw).wait()

        def edge(e, ecarry):
            srow = zero16
            for hh in range(H):
                k0 = k_v[e, pl.ds(hh * DK, 16)]
                k1 = k_v[e, pl.ds(hh * DK + 16, 16)]
                q0 = q_v[e, pl.ds(hh * DK, 16)]
                q1 = q_v[e, pl.ds(hh * DK + 16, 16)]
                p = k0 * q0 + k1 * q1
                for shift in (8, 4, 2, 1):
                    p = p + p.at[lane ^ shift].get(mode="promise_in_bounds")
                wm = jnp.exp(p * rs)
                v0 = v_v[e, pl.ds(hh * DK, 16)]
                v1 = v_v[e, pl.ds(hh * DK + 16, 16)]
                cb_v[pl.ds((2 * hh) * SLABW + e * FS, 16)] = v0 * wm
                cb_v[pl.ds((2 * hh + 1) * SLABW + e * FS, 16)] = v1 * wm
                srow = srow + jnp.where(lane == hh, wm, jnp.float32(0.0))
            cb_v[pl.ds(2 * NSUB * SLABW + e * FS, 16)] = srow
            return ecarry
        lax.fori_loop(0, B1, edge, 0)

        g = wid * NBLK1 + i
        pltpu.async_copy(cb_v, wv_hbm.at[pl.ds(g * BLKW, BLKW)], semw)
        return carry
    lax.fori_loop(0, NBLK1, eblock, 0)
    pltpu.make_async_copy(wv_hbm.at[pl.ds(0, BLKW)], cb_v, semw).wait()


def _pass2_body(dst_hbm, wv_hbm, anum_hbm, ssum_hbm,
                d_v, wv_v, ex_v, acc_v, ssa_v, semd, sem1, sem2):
    c = lax.axis_index("c")
    s = lax.axis_index("s")
    hh = s // 2
    lane = lax.iota(jnp.int32, 16)
    zero16 = jnp.zeros((16,), jnp.float32)
    even = s % 2 == 0

    def zacc(r, carry):
        acc_v[pl.ds(r * 16, 16)] = zero16
        return carry
    lax.fori_loop(0, (HALF + 8) * FS // 16, zacc, 0)

    def zss(r, carry):
        ssa_v[pl.ds(r * 16, 16)] = zero16
        return carry
    lax.fori_loop(0, (HALF + 16) // 16, zss, 0)

    def fire(slot, i):
        base = i * B2
        g0 = i * (B2 // B1)
        pltpu.async_copy(dst_hbm.at[pl.ds(base, B2)],
                         d_v.at[pl.ds(slot * B2, B2)], semd)
        for t in range(B2 // B1):
            pltpu.async_copy(
                wv_hbm.at[pl.ds((g0 + t) * BLKW + s * SLABW, SLABW)],
                wv_v.at[pl.ds(slot * B2 * FS + t * SLABW, SLABW)], sem1)

        @pl.when(even)
        def _():
            for t in range(B2 // B1):
                pltpu.async_copy(
                    wv_hbm.at[pl.ds((g0 + t) * BLKW + 2 * NSUB * SLABW,
                                    SLABW)],
                    ex_v.at[pl.ds(slot * B2 * FS + t * SLABW, SLABW)], sem2)

    fire(0, 0)

    def eblock(i, carry):
        slot = jnp.remainder(i, 2)
        sbase = slot * B2 * FS
        pltpu.make_async_copy(dst_hbm.at[pl.ds(0, B2)],
                              d_v.at[pl.ds(slot * B2, B2)], semd).wait()
        pltpu.make_async_copy(wv_hbm.at[pl.ds(0, B2 * FS)],
                              wv_v.at[pl.ds(sbase, B2 * FS)], sem1).wait()

        @pl.when(even)
        def _():
            pltpu.make_async_copy(wv_hbm.at[pl.ds(0, B2 * FS)],
                                  ex_v.at[pl.ds(sbase, B2 * FS)],
                                  sem2).wait()

        @pl.when(i < NBLK2 - 1)
        def _():
            fire(1 - slot, i + 1)

        for j in range(B2 // 16):
            d16 = d_v[pl.ds(slot * B2 + j * 16, 16)]
            lv = d16 - c * HALF
            inr = (lv >= 0) & (lv < HALF)
            lidx = jnp.where(inr, lv, DUMP)
            lbase = lidx * FS
            for f in range(FS):
                vcol = plsc.load_gather(
                    wv_v, [sbase + (lane * FS + (j * 16 * FS + f))])
                plsc.addupdate_scatter(acc_v, [lbase + f], vcol)

            @pl.when(even)
            def _():
                w16 = plsc.load_gather(
                    ex_v, [sbase + (lane * FS + (j * 16 * FS + hh))])
                plsc.addupdate_scatter(ssa_v, [lidx], w16)
        return carry
    lax.fori_loop(0, NBLK2, eblock, 0)

    wid = c * NSUB + s
    pltpu.sync_copy(acc_v.at[pl.ds(0, HALF * FS)],
                    anum_hbm.at[pl.ds(wid * HALF * FS, HALF * FS)])

    @pl.when(even)
    def _():
        off = (c * H + hh) * HALF
        pltpu.sync_copy(ssa_v.at[pl.ds(0, HALF)],
                        ssum_hbm.at[pl.ds(off, HALF)])


def _edge_phase(q, k, v, src_p, dst_p):
    mesh = plsc.VectorSubcoreMesh(core_axis_name="c", subcore_axis_name="s")
    cparams = pltpu.CompilerParams(needs_layout_passes=False)
    wv_flat = functools.partial(
        pl.kernel,
        mesh=mesh,
        compiler_params=cparams,
        out_type=jax.ShapeDtypeStruct((NBLK_G * BLKW,), jnp.float32),
        scratch_types=[
            pltpu.VMEM((EPW,), jnp.int32),
            pltpu.VMEM((EPW,), jnp.int32),
            pltpu.VMEM((EPW,), jnp.int32),
            pltpu.VMEM((B1, D), jnp.float32),
            pltpu.VMEM((B1, D), jnp.float32),
            pltpu.VMEM((B1, D), jnp.float32),
            pltpu.VMEM((BLKW,), jnp.float32),
            pltpu.SemaphoreType.DMA,
            pltpu.SemaphoreType.DMA,
            pltpu.SemaphoreType.DMA,
            pltpu.SemaphoreType.DMA,
        ],
    )(_pass1_body)(k, q, v, src_p, dst_p)

    anum_flat, ssum_flat = functools.partial(
        pl.kernel,
        mesh=mesh,
        compiler_params=cparams,
        out_type=[
            jax.ShapeDtypeStruct((NW * HALF * FS,), jnp.float32),
            jax.ShapeDtypeStruct((NCORE * H * HALF,), jnp.float32),
        ],
        scratch_types=[
            pltpu.VMEM((2 * B2,), jnp.int32),
            pltpu.VMEM((2 * B2 * FS,), jnp.float32),
            pltpu.VMEM((2 * B2 * FS,), jnp.float32),
            pltpu.VMEM(((HALF + 8) * FS,), jnp.float32),
            pltpu.VMEM((HALF + 16,), jnp.float32),
            pltpu.SemaphoreType.DMA,
            pltpu.SemaphoreType.DMA,
            pltpu.SemaphoreType.DMA,
        ],
    )(_pass2_body)(dst_p, wv_flat)
    return anum_flat, ssum_flat


def _qkv_body(h_ref, wq_ref, wk_ref, wv_ref, q_ref, k_ref, v_ref):
    hb = h_ref[...]
    q_ref[...] = jnp.dot(hb, wq_ref[...], preferred_element_type=jnp.float32)
    k_ref[...] = jnp.dot(hb, wk_ref[...], preferred_element_type=jnp.float32)
    v_ref[...] = jnp.dot(hb, wv_ref[...], preferred_element_type=jnp.float32)


def _ln(x, scale, bias):
    mu = jnp.mean(x, axis=-1, keepdims=True)
    var = jnp.mean((x - mu) ** 2, axis=-1, keepdims=True)
    return (x - mu) / jnp.sqrt(var + 1e-5) * scale + bias


def _post_body(anum_ref, ssum_ref, h_ref, exp_ref, wo_ref, l1s_ref, l1b_ref,
               w1_ref, b1_ref, w2_ref, b2_ref, l2s_ref, l2b_ref, out_ref):
    denom = jnp.dot(ssum_ref[...], exp_ref[...],
                    preferred_element_type=jnp.float32) + 1e-9
    a = anum_ref[...] / denom
    o = jnp.dot(a, wo_ref[...], preferred_element_type=jnp.float32)
    h1 = _ln(h_ref[...] + o, l1s_ref[...], l1b_ref[...])
    f = jnp.maximum(
        jnp.dot(h1, w1_ref[...], preferred_element_type=jnp.float32)
        + b1_ref[...], 0.0)
    f2 = jnp.dot(f, w2_ref[...], preferred_element_type=jnp.float32) + b2_ref[...]
    out_ref[...] = _ln(h1 + f2, l2s_ref[...], l2b_ref[...])


BLK = 1000
GRID = N // BLK


def kernel(h, edge_index, Wq, Wk, Wv, Wo, ln1_scale, ln1_bias, ln2_scale,
           ln2_bias, W1, b1, W2, b2):
    q, k, v = pl.pallas_call(
        _qkv_body,
        grid=(GRID,),
        in_specs=[
            pl.BlockSpec((BLK, D), lambda i: (i, 0)),
            pl.BlockSpec((D, D), lambda i: (0, 0)),
            pl.BlockSpec((D, D), lambda i: (0, 0)),
            pl.BlockSpec((D, D), lambda i: (0, 0)),
        ],
        out_specs=[
            pl.BlockSpec((BLK, D), lambda i: (i, 0)),
            pl.BlockSpec((BLK, D), lambda i: (i, 0)),
            pl.BlockSpec((BLK, D), lambda i: (i, 0)),
        ],
        out_shape=[
            jax.ShapeDtypeStruct((N, D), jnp.float32),
            jax.ShapeDtypeStruct((N, D), jnp.float32),
            jax.ShapeDtypeStruct((N, D), jnp.float32),
        ],
    )(h, Wq, Wk, Wv)

    src_p = jnp.pad(edge_index[0], (0, E_PAD - E))
    dst_p = jnp.pad(edge_index[1], (0, E_PAD - E), constant_values=1 << 20)

    anum_flat, ssum_flat = _edge_phase(q, k, v, src_p, dst_p)
    anum = (anum_flat.reshape(NCORE, NSUB, HALF, FS)
            .transpose(0, 2, 1, 3).reshape(NPAD, D)[:N])
    ssum = (ssum_flat.reshape(NCORE, H, HALF)
            .transpose(0, 2, 1).reshape(NPAD, H)[:N])

    expand = (jnp.arange(D, dtype=jnp.int32)[None, :] // DK
              == jnp.arange(H, dtype=jnp.int32)[:, None]).astype(jnp.float32)

    out = pl.pallas_call(
        _post_body,
        grid=(GRID,),
        in_specs=[
            pl.BlockSpec((BLK, D), lambda i: (i, 0)),
            pl.BlockSpec((BLK, H), lambda i: (i, 0)),
            pl.BlockSpec((BLK, D), lambda i: (i, 0)),
            pl.BlockSpec((H, D), lambda i: (0, 0)),
            pl.BlockSpec((D, D), lambda i: (0, 0)),
            pl.BlockSpec((1, D), lambda i: (0, 0)),
            pl.BlockSpec((1, D), lambda i: (0, 0)),
            pl.BlockSpec((D, DFF), lambda i: (0, 0)),
            pl.BlockSpec((1, DFF), lambda i: (0, 0)),
            pl.BlockSpec((DFF, D), lambda i: (0, 0)),
            pl.BlockSpec((1, D), lambda i: (0, 0)),
            pl.BlockSpec((1, D), lambda i: (0, 0)),
            pl.BlockSpec((1, D), lambda i: (0, 0)),
        ],
        out_specs=pl.BlockSpec((BLK, D), lambda i: (i, 0)),
        out_shape=jax.ShapeDtypeStruct((N, D), jnp.float32),
    )(anum, ssum, h, expand, Wo,
      ln1_scale.reshape(1, D), ln1_bias.reshape(1, D),
      W1, b1.reshape(1, DFF), W2, b2.reshape(1, D),
      ln2_scale.reshape(1, D), ln2_bias.reshape(1, D))
    return out


# speedup vs baseline: 6.3497x; 1.2490x over previous
import functools

import jax
import jax.numpy as jnp
import numpy as np
from jax import lax
from jax.experimental import pallas as pl
from jax.experimental.pallas import tpu as pltpu
from jax.experimental.pallas import tpu_sc as plsc

N = 10000
E = 160000
D = 256
H = 8
DK = 32
DFF = 1024

NPAD = 10240
HALF = NPAD // 2
NSUB = 16
NCORE = 2
NW = NSUB * NCORE
E_PAD = 163840
EPW = E_PAD // NW
B1 = 64
NBLK1 = EPW // B1
B2 = 512
NBLK2 = E_PAD // B2
FS = 16
DUMP = HALF
NSLAB = 2 * NSUB + 1
SLABW = B1 * FS
BLKW = NSLAB * SLABW
NBLK_G = E_PAD // B1


def _pass1_body(k_hbm, q_hbm, v_hbm, src_hbm, dst_hbm, wv_hbm,
                s_v, d_v, dg_v, k_v, q_v, v_v, cb_v, sem1, sem2, sem3, semw):
    c = lax.axis_index("c")
    s = lax.axis_index("s")
    wid = c * NSUB + s
    zero16 = jnp.zeros((16,), jnp.float32)
    lane = lax.iota(jnp.int32, 16)
    rs = np.float32(1.0 / np.sqrt(DK))
    ebase = wid * EPW

    pltpu.sync_copy(src_hbm.at[pl.ds(ebase, EPW)], s_v)
    pltpu.sync_copy(dst_hbm.at[pl.ds(ebase, EPW)], d_v)

    def clamp(r, carry):
        dv = d_v[pl.ds(r * 16, 16)]
        dg_v[pl.ds(r * 16, 16)] = jnp.minimum(dv, N - 1)
        return carry
    lax.fori_loop(0, EPW // 16, clamp, 0)

    def eblock(i, carry):
        base = ebase + i * B1
        cp1 = pltpu.async_copy(k_hbm.at[s_v.at[pl.ds(i * B1, B1)]],
                               k_v, sem1)
        cp2 = pltpu.async_copy(q_hbm.at[dg_v.at[pl.ds(i * B1, B1)]],
                               q_v, sem2)
        cp3 = pltpu.async_copy(v_hbm.at[s_v.at[pl.ds(i * B1, B1)]],
                               v_v, sem3)
        cp1.wait()
        cp2.wait()
        cp3.wait()
        @pl.when(i > 0)
        def _():
            pltpu.make_async_copy(
                wv_hbm.at[pl.ds(0, BLKW)], cb_v, semw).wait()

        def edge(e, ecarry):
            srow = zero16
            for hh in range(H):
                k0 = k_v[e, pl.ds(hh * DK, 16)]
                k1 = k_v[e, pl.ds(hh * DK + 16, 16)]
                q0 = q_v[e, pl.ds(hh * DK, 16)]
                q1 = q_v[e, pl.ds(hh * DK + 16, 16)]
                p = k0 * q0 + k1 * q1
                for shift in (8, 4, 2, 1):
                    p = p + p.at[lane ^ shift].get(mode="promise_in_bounds")
                wm = jnp.exp(p * rs)
                v0 = v_v[e, pl.ds(hh * DK, 16)]
                v1 = v_v[e, pl.ds(hh * DK + 16, 16)]
                cb_v[pl.ds((2 * hh) * SLABW + e * FS, 16)] = v0 * wm
                cb_v[pl.ds((2 * hh + 1) * SLABW + e * FS, 16)] = v1 * wm
                srow = srow + jnp.where(lane == hh, wm, jnp.float32(0.0))
            cb_v[pl.ds(2 * NSUB * SLABW + e * FS, 16)] = srow
            return ecarry
        lax.fori_loop(0, B1, edge, 0)

        g = wid * NBLK1 + i
        pltpu.async_copy(cb_v, wv_hbm.at[pl.ds(g * BLKW, BLKW)], semw)
        return carry
    lax.fori_loop(0, NBLK1, eblock, 0)
    pltpu.make_async_copy(wv_hbm.at[pl.ds(0, BLKW)], cb_v, semw).wait()


def _pass2_body(dst_hbm, wv_hbm, anum_hbm, ssum_hbm,
                d_v, wv_v, ex_v, acc_v, ssa_v, semd, sem1, sem2):
    c = lax.axis_index("c")
    s = lax.axis_index("s")
    hh = s // 2
    lane = lax.iota(jnp.int32, 16)
    zero16 = jnp.zeros((16,), jnp.float32)
    even = s % 2 == 0

    def zacc(r, carry):
        acc_v[pl.ds(r * 16, 16)] = zero16
        return carry
    lax.fori_loop(0, (HALF + 8) * FS // 16, zacc, 0)

    def zss(r, carry):
        ssa_v[pl.ds(r * 16, 16)] = zero16
        return carry
    lax.fori_loop(0, (HALF + 16) // 16, zss, 0)

    def fire(slot, i):
        base = i * B2
        g0 = i * (B2 // B1)
        pltpu.async_copy(dst_hbm.at[pl.ds(base, B2)],
                         d_v.at[pl.ds(slot * B2, B2)], semd)
        for t in range(B2 // B1):
            pltpu.async_copy(
                wv_hbm.at[pl.ds((g0 + t) * BLKW + s * SLABW, SLABW)],
                wv_v.at[pl.ds(slot * B2 * FS + t * SLABW, SLABW)], sem1)

        @pl.when(even)
        def _():
            for t in range(B2 // B1):
                pltpu.async_copy(
                    wv_hbm.at[pl.ds((g0 + t) * BLKW + 2 * NSUB * SLABW,
                                    SLABW)],
                    ex_v.at[pl.ds(slot * B2 * FS + t * SLABW, SLABW)], sem2)

    fire(0, 0)

    def eblock(i, carry):
        slot = jnp.remainder(i, 2)
        sbase = slot * B2 * FS
        pltpu.make_async_copy(dst_hbm.at[pl.ds(0, B2)],
                              d_v.at[pl.ds(slot * B2, B2)], semd).wait()
        pltpu.make_async_copy(wv_hbm.at[pl.ds(0, B2 * FS)],
                              wv_v.at[pl.ds(sbase, B2 * FS)], sem1).wait()

        @pl.when(even)
        def _():
            pltpu.make_async_copy(wv_hbm.at[pl.ds(0, B2 * FS)],
                                  ex_v.at[pl.ds(sbase, B2 * FS)],
                                  sem2).wait()

        @pl.when(i < NBLK2 - 1)
        def _():
            fire(1 - slot, i + 1)

        for j in range(B2 // 16):
            d16 = d_v[pl.ds(slot * B2 + j * 16, 16)]
            lv = d16 - c * HALF
            inr = (lv >= 0) & (lv < HALF)
            lidx = jnp.where(inr, lv, DUMP)
            lbase = lidx * FS
            vcols = [plsc.load_gather(
                wv_v, [sbase + (lane * FS + (j * 16 * FS + f))])
                for f in range(FS)]
            for f in range(FS):
                plsc.addupdate_scatter(acc_v, [lbase + f], vcols[f])

            @pl.when(even)
            def _():
                w16 = plsc.load_gather(
                    ex_v, [sbase + (lane * FS + (j * 16 * FS + hh))])
                plsc.addupdate_scatter(ssa_v, [lidx], w16)
        return carry
    lax.fori_loop(0, NBLK2, eblock, 0)

    wid = c * NSUB + s
    pltpu.sync_copy(acc_v.at[pl.ds(0, HALF * FS)],
                    anum_hbm.at[pl.ds(wid * HALF * FS, HALF * FS)])

    @pl.when(even)
    def _():
        off = (c * H + hh) * HALF
        pltpu.sync_copy(ssa_v.at[pl.ds(0, HALF)],
                        ssum_hbm.at[pl.ds(off, HALF)])


def _edge_phase(q, k, v, src_p, dst_p):
    mesh = plsc.VectorSubcoreMesh(core_axis_name="c", subcore_axis_name="s")
    cparams = pltpu.CompilerParams(needs_layout_passes=False)
    wv_flat = functools.partial(
        pl.kernel,
        mesh=mesh,
        compiler_params=cparams,
        out_type=jax.ShapeDtypeStruct((NBLK_G * BLKW,), jnp.float32),
        scratch_types=[
            pltpu.VMEM((EPW,), jnp.int32),
            pltpu.VMEM((EPW,), jnp.int32),
            pltpu.VMEM((EPW,), jnp.int32),
            pltpu.VMEM((B1, D), jnp.float32),
            pltpu.VMEM((B1, D), jnp.float32),
            pltpu.VMEM((B1, D), jnp.float32),
            pltpu.VMEM((BLKW,), jnp.float32),
            pltpu.SemaphoreType.DMA,
            pltpu.SemaphoreType.DMA,
            pltpu.SemaphoreType.DMA,
            pltpu.SemaphoreType.DMA,
        ],
    )(_pass1_body)(k, q, v, src_p, dst_p)

    anum_flat, ssum_flat = functools.partial(
        pl.kernel,
        mesh=mesh,
        compiler_params=cparams,
        out_type=[
            jax.ShapeDtypeStruct((NW * HALF * FS,), jnp.float32),
            jax.ShapeDtypeStruct((NCORE * H * HALF,), jnp.float32),
        ],
        scratch_types=[
            pltpu.VMEM((2 * B2,), jnp.int32),
            pltpu.VMEM((2 * B2 * FS,), jnp.float32),
            pltpu.VMEM((2 * B2 * FS,), jnp.float32),
            pltpu.VMEM(((HALF + 8) * FS,), jnp.float32),
            pltpu.VMEM((HALF + 16,), jnp.float32),
            pltpu.SemaphoreType.DMA,
            pltpu.SemaphoreType.DMA,
            pltpu.SemaphoreType.DMA,
        ],
    )(_pass2_body)(dst_p, wv_flat)
    return anum_flat, ssum_flat


def _qkv_body(h_ref, wq_ref, wk_ref, wv_ref, q_ref, k_ref, v_ref):
    hb = h_ref[...]
    q_ref[...] = jnp.dot(hb, wq_ref[...], preferred_element_type=jnp.float32)
    k_ref[...] = jnp.dot(hb, wk_ref[...], preferred_element_type=jnp.float32)
    v_ref[...] = jnp.dot(hb, wv_ref[...], preferred_element_type=jnp.float32)


def _ln(x, scale, bias):
    mu = jnp.mean(x, axis=-1, keepdims=True)
    var = jnp.mean((x - mu) ** 2, axis=-1, keepdims=True)
    return (x - mu) / jnp.sqrt(var + 1e-5) * scale + bias


def _post_body(anum_ref, ssum_ref, h_ref, exp_ref, wo_ref, l1s_ref, l1b_ref,
               w1_ref, b1_ref, w2_ref, b2_ref, l2s_ref, l2b_ref, out_ref):
    denom = jnp.dot(ssum_ref[...], exp_ref[...],
                    preferred_element_type=jnp.float32) + 1e-9
    a = anum_ref[...] / denom
    o = jnp.dot(a, wo_ref[...], preferred_element_type=jnp.float32)
    h1 = _ln(h_ref[...] + o, l1s_ref[...], l1b_ref[...])
    f = jnp.maximum(
        jnp.dot(h1, w1_ref[...], preferred_element_type=jnp.float32)
        + b1_ref[...], 0.0)
    f2 = jnp.dot(f, w2_ref[...], preferred_element_type=jnp.float32) + b2_ref[...]
    out_ref[...] = _ln(h1 + f2, l2s_ref[...], l2b_ref[...])


BLK = 1000
GRID = N // BLK


def kernel(h, edge_index, Wq, Wk, Wv, Wo, ln1_scale, ln1_bias, ln2_scale,
           ln2_bias, W1, b1, W2, b2):
    q, k, v = pl.pallas_call(
        _qkv_body,
        grid=(GRID,),
        in_specs=[
            pl.BlockSpec((BLK, D), lambda i: (i, 0)),
            pl.BlockSpec((D, D), lambda i: (0, 0)),
            pl.BlockSpec((D, D), lambda i: (0, 0)),
            pl.BlockSpec((D, D), lambda i: (0, 0)),
        ],
        out_specs=[
            pl.BlockSpec((BLK, D), lambda i: (i, 0)),
            pl.BlockSpec((BLK, D), lambda i: (i, 0)),
            pl.BlockSpec((BLK, D), lambda i: (i, 0)),
        ],
        out_shape=[
            jax.ShapeDtypeStruct((N, D), jnp.float32),
            jax.ShapeDtypeStruct((N, D), jnp.float32),
            jax.ShapeDtypeStruct((N, D), jnp.float32),
        ],
    )(h, Wq, Wk, Wv)

    src_p = jnp.pad(edge_index[0], (0, E_PAD - E))
    dst_p = jnp.pad(edge_index[1], (0, E_PAD - E), constant_values=1 << 20)

    anum_flat, ssum_flat = _edge_phase(q, k, v, src_p, dst_p)
    anum = (anum_flat.reshape(NCORE, NSUB, HALF, FS)
            .transpose(0, 2, 1, 3).reshape(NPAD, D)[:N])
    ssum = (ssum_flat.reshape(NCORE, H, HALF)
            .transpose(0, 2, 1).reshape(NPAD, H)[:N])

    expand = (jnp.arange(D, dtype=jnp.int32)[None, :] // DK
              == jnp.arange(H, dtype=jnp.int32)[:, None]).astype(jnp.float32)

    out = pl.pallas_call(
        _post_body,
        grid=(GRID,),
        in_specs=[
            pl.BlockSpec((BLK, D), lambda i: (i, 0)),
            pl.BlockSpec((BLK, H), lambda i: (i, 0)),
            pl.BlockSpec((BLK, D), lambda i: (i, 0)),
            pl.BlockSpec((H, D), lambda i: (0, 0)),
            pl.BlockSpec((D, D), lambda i: (0, 0)),
            pl.BlockSpec((1, D), lambda i: (0, 0)),
            pl.BlockSpec((1, D), lambda i: (0, 0)),
            pl.BlockSpec((D, DFF), lambda i: (0, 0)),
            pl.BlockSpec((1, DFF), lambda i: (0, 0)),
            pl.BlockSpec((DFF, D), lambda i: (0, 0)),
            pl.BlockSpec((1, D), lambda i: (0, 0)),
            pl.BlockSpec((1, D), lambda i: (0, 0)),
            pl.BlockSpec((1, D), lambda i: (0, 0)),
        ],
        out_specs=pl.BlockSpec((BLK, D), lambda i: (i, 0)),
        out_shape=jax.ShapeDtypeStruct((N, D), jnp.float32),
    )(anum, ssum, h, expand, Wo,
      ln1_scale.reshape(1, D), ln1_bias.reshape(1, D),
      W1, b1.reshape(1, DFF), W2, b2.reshape(1, D),
      ln2_scale.reshape(1, D), ln2_bias.reshape(1, D))
    return out


# speedup vs baseline: 7.9304x; 1.2489x over previous
import functools

import jax
import jax.numpy as jnp
import numpy as np
from jax import lax
from jax.experimental import pallas as pl
from jax.experimental.pallas import tpu as pltpu
from jax.experimental.pallas import tpu_sc as plsc

N = 10000
E = 160000
D = 256
H = 8
DK = 32
DFF = 1024

NPAD = 10240
HALF = NPAD // 2
NSUB = 16
NCORE = 2
NW = NSUB * NCORE
E_PAD = 163840
EPW = E_PAD // NW
B1 = 64
NBLK1 = EPW // B1
B2 = 512
NBLK2 = E_PAD // B2
FS = 16
DUMP = HALF
NSLAB = 2 * NSUB + 1
SLABW = B1 * FS
BLKW = NSLAB * SLABW
NBLK_G = E_PAD // B1


def _pass1_body(k_hbm, q_hbm, v_hbm, src_hbm, dst_hbm, wv_hbm,
                s_v, d_v, dg_v, k_v, q_v, v_v, cb_v, sem1, sem2, sem3, semw):
    c = lax.axis_index("c")
    s = lax.axis_index("s")
    wid = c * NSUB + s
    zero16 = jnp.zeros((16,), jnp.float32)
    lane = lax.iota(jnp.int32, 16)
    rs = np.float32(1.0 / np.sqrt(DK))
    ebase = wid * EPW

    pltpu.sync_copy(src_hbm.at[pl.ds(ebase, EPW)], s_v)
    pltpu.sync_copy(dst_hbm.at[pl.ds(ebase, EPW)], d_v)

    def clamp(r, carry):
        dv = d_v[pl.ds(r * 16, 16)]
        dg_v[pl.ds(r * 16, 16)] = jnp.minimum(dv, N - 1)
        return carry
    lax.fori_loop(0, EPW // 16, clamp, 0)

    def eblock(i, carry):
        base = ebase + i * B1
        cp1 = pltpu.async_copy(k_hbm.at[s_v.at[pl.ds(i * B1, B1)]],
                               k_v, sem1)
        cp2 = pltpu.async_copy(q_hbm.at[dg_v.at[pl.ds(i * B1, B1)]],
                               q_v, sem2)
        cp3 = pltpu.async_copy(v_hbm.at[s_v.at[pl.ds(i * B1, B1)]],
                               v_v, sem3)
        cp1.wait()
        cp2.wait()
        cp3.wait()
        @pl.when(i > 0)
        def _():
            pltpu.make_async_copy(
                wv_hbm.at[pl.ds(0, BLKW)], cb_v, semw).wait()

        def edge(e, ecarry):
            ps = []
            for hh in range(H):
                k0 = k_v[e, pl.ds(hh * DK, 16)]
                k1 = k_v[e, pl.ds(hh * DK + 16, 16)]
                q0 = q_v[e, pl.ds(hh * DK, 16)]
                q1 = q_v[e, pl.ds(hh * DK + 16, 16)]
                ps.append(k0 * q0 + k1 * q1)
            for shift in (8, 4, 2, 1):
                ps = [p + p.at[lane ^ shift].get(mode="promise_in_bounds")
                      for p in ps]
            wms = [jnp.exp(p * rs) for p in ps]
            srow = zero16
            for hh in range(H):
                v0 = v_v[e, pl.ds(hh * DK, 16)]
                v1 = v_v[e, pl.ds(hh * DK + 16, 16)]
                cb_v[pl.ds((2 * hh) * SLABW + e * FS, 16)] = v0 * wms[hh]
                cb_v[pl.ds((2 * hh + 1) * SLABW + e * FS, 16)] = v1 * wms[hh]
                srow = srow + jnp.where(lane == hh, wms[hh], jnp.float32(0.0))
            cb_v[pl.ds(2 * NSUB * SLABW + e * FS, 16)] = srow
            return ecarry
        lax.fori_loop(0, B1, edge, 0)

        g = wid * NBLK1 + i
        pltpu.async_copy(cb_v, wv_hbm.at[pl.ds(g * BLKW, BLKW)], semw)
        return carry
    lax.fori_loop(0, NBLK1, eblock, 0)
    pltpu.make_async_copy(wv_hbm.at[pl.ds(0, BLKW)], cb_v, semw).wait()


def _pass2_body(dst_hbm, wv_hbm, anum_hbm, ssum_hbm,
                d_v, wv_v, ex_v, acc_v, ssa_v, semd, sem1, sem2):
    c = lax.axis_index("c")
    s = lax.axis_index("s")
    hh = s // 2
    lane = lax.iota(jnp.int32, 16)
    zero16 = jnp.zeros((16,), jnp.float32)
    even = s % 2 == 0

    def zacc(r, carry):
        acc_v[pl.ds(r * 16, 16)] = zero16
        return carry
    lax.fori_loop(0, (HALF + 8) * FS // 16, zacc, 0)

    def zss(r, carry):
        ssa_v[pl.ds(r * 16, 16)] = zero16
        return carry
    lax.fori_loop(0, (HALF + 16) // 16, zss, 0)

    def fire(slot, i):
        base = i * B2
        g0 = i * (B2 // B1)
        pltpu.async_copy(dst_hbm.at[pl.ds(base, B2)],
                         d_v.at[pl.ds(slot * B2, B2)], semd)
        for t in range(B2 // B1):
            pltpu.async_copy(
                wv_hbm.at[pl.ds((g0 + t) * BLKW + s * SLABW, SLABW)],
                wv_v.at[pl.ds(slot * B2 * FS + t * SLABW, SLABW)], sem1)

        @pl.when(even)
        def _():
            for t in range(B2 // B1):
                pltpu.async_copy(
                    wv_hbm.at[pl.ds((g0 + t) * BLKW + 2 * NSUB * SLABW,
                                    SLABW)],
                    ex_v.at[pl.ds(slot * B2 * FS + t * SLABW, SLABW)], sem2)

    fire(0, 0)

    def eblock(i, carry):
        slot = jnp.remainder(i, 2)
        sbase = slot * B2 * FS
        pltpu.make_async_copy(dst_hbm.at[pl.ds(0, B2)],
                              d_v.at[pl.ds(slot * B2, B2)], semd).wait()
        pltpu.make_async_copy(wv_hbm.at[pl.ds(0, B2 * FS)],
                              wv_v.at[pl.ds(sbase, B2 * FS)], sem1).wait()

        @pl.when(even)
        def _():
            pltpu.make_async_copy(wv_hbm.at[pl.ds(0, B2 * FS)],
                                  ex_v.at[pl.ds(sbase, B2 * FS)],
                                  sem2).wait()

        @pl.when(i < NBLK2 - 1)
        def _():
            fire(1 - slot, i + 1)

        for j in range(B2 // 16):
            d16 = d_v[pl.ds(slot * B2 + j * 16, 16)]
            lv = d16 - c * HALF
            inr = (lv >= 0) & (lv < HALF)
            lidx = jnp.where(inr, lv, DUMP)
            lbase = lidx * FS
            vcols = [plsc.load_gather(
                wv_v, [sbase + (lane * FS + (j * 16 * FS + f))])
                for f in range(FS)]
            for f in range(FS):
                plsc.addupdate_scatter(acc_v, [lbase + f], vcols[f])

            @pl.when(even)
            def _():
                w16 = plsc.load_gather(
                    ex_v, [sbase + (lane * FS + (j * 16 * FS + hh))])
                plsc.addupdate_scatter(ssa_v, [lidx], w16)
        return carry
    lax.fori_loop(0, NBLK2, eblock, 0)

    wid = c * NSUB + s
    pltpu.sync_copy(acc_v.at[pl.ds(0, HALF * FS)],
                    anum_hbm.at[pl.ds(wid * HALF * FS, HALF * FS)])

    @pl.when(even)
    def _():
        off = (c * H + hh) * HALF
        pltpu.sync_copy(ssa_v.at[pl.ds(0, HALF)],
                        ssum_hbm.at[pl.ds(off, HALF)])


def _edge_phase(q, k, v, src_p, dst_p):
    mesh = plsc.VectorSubcoreMesh(core_axis_name="c", subcore_axis_name="s")
    cparams = pltpu.CompilerParams(needs_layout_passes=False)
    wv_flat = functools.partial(
        pl.kernel,
        mesh=mesh,
        compiler_params=cparams,
        out_type=jax.ShapeDtypeStruct((NBLK_G * BLKW,), jnp.float32),
        scratch_types=[
            pltpu.VMEM((EPW,), jnp.int32),
            pltpu.VMEM((EPW,), jnp.int32),
            pltpu.VMEM((EPW,), jnp.int32),
            pltpu.VMEM((B1, D), jnp.float32),
            pltpu.VMEM((B1, D), jnp.float32),
            pltpu.VMEM((B1, D), jnp.float32),
            pltpu.VMEM((BLKW,), jnp.float32),
            pltpu.SemaphoreType.DMA,
            pltpu.SemaphoreType.DMA,
            pltpu.SemaphoreType.DMA,
            pltpu.SemaphoreType.DMA,
        ],
    )(_pass1_body)(k, q, v, src_p, dst_p)

    anum_flat, ssum_flat = functools.partial(
        pl.kernel,
        mesh=mesh,
        compiler_params=cparams,
        out_type=[
            jax.ShapeDtypeStruct((NW * HALF * FS,), jnp.float32),
            jax.ShapeDtypeStruct((NCORE * H * HALF,), jnp.float32),
        ],
        scratch_types=[
            pltpu.VMEM((2 * B2,), jnp.int32),
            pltpu.VMEM((2 * B2 * FS,), jnp.float32),
            pltpu.VMEM((2 * B2 * FS,), jnp.float32),
            pltpu.VMEM(((HALF + 8) * FS,), jnp.float32),
            pltpu.VMEM((HALF + 16,), jnp.float32),
            pltpu.SemaphoreType.DMA,
            pltpu.SemaphoreType.DMA,
            pltpu.SemaphoreType.DMA,
        ],
    )(_pass2_body)(dst_p, wv_flat)
    return anum_flat, ssum_flat


def _qkv_body(h_ref, wq_ref, wk_ref, wv_ref, q_ref, k_ref, v_ref):
    hb = h_ref[...]
    q_ref[...] = jnp.dot(hb, wq_ref[...], preferred_element_type=jnp.float32)
    k_ref[...] = jnp.dot(hb, wk_ref[...], preferred_element_type=jnp.float32)
    v_ref[...] = jnp.dot(hb, wv_ref[...], preferred_element_type=jnp.float32)


def _ln(x, scale, bias):
    mu = jnp.mean(x, axis=-1, keepdims=True)
    var = jnp.mean((x - mu) ** 2, axis=-1, keepdims=True)
    return (x - mu) / jnp.sqrt(var + 1e-5) * scale + bias


def _post_body(anum_ref, ssum_ref, h_ref, exp_ref, wo_ref, l1s_ref, l1b_ref,
               w1_ref, b1_ref, w2_ref, b2_ref, l2s_ref, l2b_ref, out_ref):
    denom = jnp.dot(ssum_ref[...], exp_ref[...],
                    preferred_element_type=jnp.float32) + 1e-9
    a = anum_ref[...] / denom
    o = jnp.dot(a, wo_ref[...], preferred_element_type=jnp.float32)
    h1 = _ln(h_ref[...] + o, l1s_ref[...], l1b_ref[...])
    f = jnp.maximum(
        jnp.dot(h1, w1_ref[...], preferred_element_type=jnp.float32)
        + b1_ref[...], 0.0)
    f2 = jnp.dot(f, w2_ref[...], preferred_element_type=jnp.float32) + b2_ref[...]
    out_ref[...] = _ln(h1 + f2, l2s_ref[...], l2b_ref[...])


BLK = 1000
GRID = N // BLK


def kernel(h, edge_index, Wq, Wk, Wv, Wo, ln1_scale, ln1_bias, ln2_scale,
           ln2_bias, W1, b1, W2, b2):
    q, k, v = pl.pallas_call(
        _qkv_body,
        grid=(GRID,),
        in_specs=[
            pl.BlockSpec((BLK, D), lambda i: (i, 0)),
            pl.BlockSpec((D, D), lambda i: (0, 0)),
            pl.BlockSpec((D, D), lambda i: (0, 0)),
            pl.BlockSpec((D, D), lambda i: (0, 0)),
        ],
        out_specs=[
            pl.BlockSpec((BLK, D), lambda i: (i, 0)),
            pl.BlockSpec((BLK, D), lambda i: (i, 0)),
            pl.BlockSpec((BLK, D), lambda i: (i, 0)),
        ],
        out_shape=[
            jax.ShapeDtypeStruct((N, D), jnp.float32),
            jax.ShapeDtypeStruct((N, D), jnp.float32),
            jax.ShapeDtypeStruct((N, D), jnp.float32),
        ],
    )(h, Wq, Wk, Wv)

    src_p = jnp.pad(edge_index[0], (0, E_PAD - E))
    dst_p = jnp.pad(edge_index[1], (0, E_PAD - E), constant_values=1 << 20)

    anum_flat, ssum_flat = _edge_phase(q, k, v, src_p, dst_p)
    anum = (anum_flat.reshape(NCORE, NSUB, HALF, FS)
            .transpose(0, 2, 1, 3).reshape(NPAD, D)[:N])
    ssum = (ssum_flat.reshape(NCORE, H, HALF)
            .transpose(0, 2, 1).reshape(NPAD, H)[:N])

    expand = (jnp.arange(D, dtype=jnp.int32)[None, :] // DK
              == jnp.arange(H, dtype=jnp.int32)[:, None]).astype(jnp.float32)

    out = pl.pallas_call(
        _post_body,
        grid=(GRID,),
        in_specs=[
            pl.BlockSpec((BLK, D), lambda i: (i, 0)),
            pl.BlockSpec((BLK, H), lambda i: (i, 0)),
            pl.BlockSpec((BLK, D), lambda i: (i, 0)),
            pl.BlockSpec((H, D), lambda i: (0, 0)),
            pl.BlockSpec((D, D), lambda i: (0, 0)),
            pl.BlockSpec((1, D), lambda i: (0, 0)),
            pl.BlockSpec((1, D), lambda i: (0, 0)),
            pl.BlockSpec((D, DFF), lambda i: (0, 0)),
            pl.BlockSpec((1, DFF), lambda i: (0, 0)),
            pl.BlockSpec((DFF, D), lambda i: (0, 0)),
            pl.BlockSpec((1, D), lambda i: (0, 0)),
            pl.BlockSpec((1, D), lambda i: (0, 0)),
            pl.BlockSpec((1, D), lambda i: (0, 0)),
        ],
        out_specs=pl.BlockSpec((BLK, D), lambda i: (i, 0)),
        out_shape=jax.ShapeDtypeStruct((N, D), jnp.float32),
    )(anum, ssum, h, expand, Wo,
      ln1_scale.reshape(1, D), ln1_bias.reshape(1, D),
      W1, b1.reshape(1, DFF), W2, b2.reshape(1, D),
      ln2_scale.reshape(1, D), ln2_bias.reshape(1, D))
    return out
